# baseline (speedup 1.0000x reference)
import functools

import jax
import jax.numpy as jnp
from jax import lax
from jax.experimental import pallas as pl
from jax.experimental.pallas import tpu as pltpu
from jax.experimental.pallas import tpu_sc as plsc

_N_E = 8192
_E_DIM = 256
_BETA = 0.25

_BM = 256
_BN = 2048


def _dist_argmin_body(z_ref, w_ref, rs_ref, cs_ref, idx_ref, rmin_ref, ridx_ref):
    n = pl.program_id(0)
    m = pl.program_id(1)
    msl = pl.ds(m * _BM, _BM)

    @pl.when(n == 0)
    def _init():
        rmin_ref[msl, :] = jnp.full((_BM, 1), jnp.inf, jnp.float32)
        ridx_ref[msl, :] = jnp.zeros((_BM, 1), jnp.int32)

    mm2 = lax.dot_general(
        z_ref[...], w_ref[...],
        dimension_numbers=(((1,), (1,)), ((), ())),
        preferred_element_type=jnp.float32,
    )
    d2 = (rs_ref[...] + cs_ref[...]) - mm2
    zd = jnp.sqrt(jnp.maximum(d2, 0.0))
    big = jnp.int32(2**31 - 1)
    v = zd[:, :128]
    ci = jnp.zeros((_BM, 128), jnp.int32)
    for c in range(1, _BN // 128):
        nv = zd[:, c * 128:(c + 1) * 128]
        lt = nv < v
        v = jnp.where(lt, nv, v)
        ci = jnp.where(lt, jnp.int32(c), ci)
    bmin = jnp.min(v, axis=1, keepdims=True)
    m1 = v == bmin
    cmin = jnp.min(jnp.where(m1, ci, big), axis=1, keepdims=True)
    m2 = m1 & (ci == cmin)
    lane = lax.broadcasted_iota(jnp.int32, (_BM, 128), 1)
    lidx = jnp.min(jnp.where(m2, lane, big), axis=1, keepdims=True)
    bidx = n * _BN + cmin * 128 + lidx
    prev = rmin_ref[msl, :]
    better = bmin < prev
    bq = bmin.astype(jnp.bfloat16).astype(jnp.float32)
    rmin_ref[msl, :] = jnp.where(better, bq, prev)
    ridx_ref[msl, :] = jnp.where(better, bidx, ridx_ref[msl, :])

    @pl.when(n == pl.num_programs(0) - 1)
    def _emit():
        idx_ref[...] = ridx_ref[msl, :]


def _dist_argmin(z_flat, W, rs, cs):
    grid = (_N_E // _BN, z_flat.shape[0] // _BM)
    return pl.pallas_call(
        _dist_argmin_body,
        grid=grid,
        in_specs=[
            pl.BlockSpec((_BM, _E_DIM), lambda n, m: (m, 0)),
            pl.BlockSpec((_BN, _E_DIM), lambda n, m: (n, 0)),
            pl.BlockSpec((_BM, 1), lambda n, m: (m, 0)),
            pl.BlockSpec((1, _BN), lambda n, m: (0, n)),
        ],
        out_specs=pl.BlockSpec((_BM, 1), lambda n, m: (m, 0)),
        out_shape=jax.ShapeDtypeStruct((z_flat.shape[0], 1), jnp.int32),
        scratch_shapes=[
            pltpu.VMEM((z_flat.shape[0], 1), jnp.float32),
            pltpu.VMEM((z_flat.shape[0], 1), jnp.int32),
        ],
        compiler_params=pltpu.CompilerParams(
            dimension_semantics=("arbitrary", "arbitrary"),
        ),
    )(z_flat, W, rs, cs)


def _make_sc_gather(B, D):
    info = plsc.get_sparse_core_info()
    nw = info.num_cores * info.num_subcores
    b_per_w = B // nw
    mesh = plsc.VectorSubcoreMesh(core_axis_name="c", subcore_axis_name="s")

    @functools.partial(
        pl.kernel, mesh=mesh,
        out_type=jax.ShapeDtypeStruct((B, D), jnp.float32),
        scratch_types=[
            pltpu.VMEM((b_per_w,), jnp.int32),
            pltpu.VMEM((b_per_w, D), jnp.float32),
            pltpu.SemaphoreType.DMA,
        ],
    )
    def gather_rows(idx_hbm, table_hbm, out_hbm, idx_v, rows_v, sem):
        wid = lax.axis_index("s") * info.num_cores + lax.axis_index("c")
        base = wid * b_per_w
        pltpu.sync_copy(idx_hbm.at[pl.ds(base, b_per_w)], idx_v)
        pltpu.async_copy(table_hbm.at[idx_v], rows_v, sem).wait()
        pltpu.sync_copy(rows_v, out_hbm.at[pl.ds(base, b_per_w)])

    return gather_rows


def _st_loss_body(z_ref, zq_ref, st_ref, tot_ref, acc_ref):
    m = pl.program_id(0)

    @pl.when(m == 0)
    def _init():
        acc_ref[0] = 0.0

    diff = zq_ref[...] - z_ref[...]
    st_ref[...] = z_ref[...] + diff
    acc_ref[0] += jnp.sum(diff * diff)

    @pl.when(m == pl.num_programs(0) - 1)
    def _emit():
        tot_ref[0, 0] = acc_ref[0]


def _st_and_loss(z_flat, z_q):
    rows = z_flat.shape[0]
    bm = 1024
    grid = (rows // bm,)
    return pl.pallas_call(
        _st_loss_body,
        grid=grid,
        in_specs=[
            pl.BlockSpec((bm, _E_DIM), lambda m: (m, 0)),
            pl.BlockSpec((bm, _E_DIM), lambda m: (m, 0)),
        ],
        out_specs=[
            pl.BlockSpec((bm, _E_DIM), lambda m: (m, 0)),
            pl.BlockSpec((1, 1), lambda m: (0, 0), memory_space=pltpu.SMEM),
        ],
        out_shape=[
            jax.ShapeDtypeStruct((rows, _E_DIM), jnp.float32),
            jax.ShapeDtypeStruct((1, 1), jnp.float32),
        ],
        scratch_shapes=[pltpu.SMEM((1,), jnp.float32)],
        compiler_params=pltpu.CompilerParams(
            dimension_semantics=("arbitrary",),
        ),
    )(z_flat, z_q)


def kernel(z, W):
    z_flat = z.reshape(-1, _E_DIM)
    rs = jnp.sum(z ** 2, axis=-1).reshape(-1, 1)
    cs = jnp.sum(W ** 2, axis=1)[None, :]
    z2 = (2.0 * z_flat).astype(jnp.bfloat16)
    idx2d = _dist_argmin(z2, W, rs, cs)
    idx = idx2d.reshape(-1)
    z_q = _make_sc_gather(z_flat.shape[0], _E_DIM)(idx, W)
    st, total = _st_and_loss(z_flat, z_q)
    mse = total[0, 0] / z.size
    loss = (_BETA * mse, mse)
    return st.reshape(z.shape), loss, idx


# speedup vs baseline: 1.2483x; 1.2483x over previous
import functools

import jax
import jax.numpy as jnp
from jax import lax
from jax.experimental import pallas as pl
from jax.experimental.pallas import tpu as pltpu
from jax.experimental.pallas import tpu_sc as plsc

_N_E = 8192
_E_DIM = 256
_BETA = 0.25

_BM = 512
_BN = 2048


def _dist_argmin_body(z_ref, w_ref, rs_ref, cs_ref, idx_ref, rmin_ref, ridx_ref):
    n = pl.program_id(0)
    m = pl.program_id(1)
    msl = pl.ds(m * _BM, _BM)

    @pl.when(n == 0)
    def _init():
        rmin_ref[msl, :] = jnp.full((_BM, 1), jnp.inf, jnp.float32)
        ridx_ref[msl, :] = jnp.zeros((_BM, 1), jnp.int32)

    mm2 = lax.dot_general(
        z_ref[...], w_ref[...],
        dimension_numbers=(((1,), (1,)), ((), ())),
        preferred_element_type=jnp.float32,
    )
    rs = rs_ref[...]
    v = None
    for c in range(_BN // 128):
        csc = cs_ref[:, c * 128:(c + 1) * 128]
        d2 = (rs + csc) - mm2[:, c * 128:(c + 1) * 128]
        dc = jnp.maximum(d2, 0.0)
        zdc = jnp.sqrt(dc)
        if v is None:
            v = zdc
            ci = jnp.zeros((_BM, 128), jnp.float32)
        else:
            lt = zdc < v
            v = jnp.where(lt, zdc, v)
            ci = jnp.where(lt, jnp.float32(c * 128), ci)
    bmin = jnp.min(v, axis=1, keepdims=True)
    lane = lax.broadcasted_iota(jnp.int32, (_BM, 128), 1).astype(jnp.float32)
    jf = jnp.min(
        jnp.where(v == bmin, ci + lane, jnp.float32(3.0e38)),
        axis=1, keepdims=True,
    )
    bidx = n * _BN + jf.astype(jnp.int32)
    prev = rmin_ref[msl, :]
    better = bmin < prev
    bq = bmin.astype(jnp.bfloat16).astype(jnp.float32)
    rmin_ref[msl, :] = jnp.where(better, bq, prev)
    ridx_ref[msl, :] = jnp.where(better, bidx, ridx_ref[msl, :])

    @pl.when(n == pl.num_programs(0) - 1)
    def _emit():
        idx_ref[...] = ridx_ref[msl, :]


def _dist_argmin(z_flat, W, rs, cs):
    grid = (_N_E // _BN, z_flat.shape[0] // _BM)
    return pl.pallas_call(
        _dist_argmin_body,
        grid=grid,
        in_specs=[
            pl.BlockSpec((_BM, _E_DIM), lambda n, m: (m, 0)),
            pl.BlockSpec((_BN, _E_DIM), lambda n, m: (n, 0)),
            pl.BlockSpec((_BM, 1), lambda n, m: (m, 0)),
            pl.BlockSpec((1, _BN), lambda n, m: (0, n)),
        ],
        out_specs=pl.BlockSpec((_BM, 1), lambda n, m: (m, 0)),
        out_shape=jax.ShapeDtypeStruct((z_flat.shape[0], 1), jnp.int32),
        scratch_shapes=[
            pltpu.VMEM((z_flat.shape[0], 1), jnp.float32),
            pltpu.VMEM((z_flat.shape[0], 1), jnp.int32),
        ],
        compiler_params=pltpu.CompilerParams(
            dimension_semantics=("arbitrary", "arbitrary"),
        ),
    )(z_flat, W, rs, cs)


def _make_sc_gather(B, D):
    info = plsc.get_sparse_core_info()
    nw = info.num_cores * info.num_subcores
    b_per_w = B // nw
    mesh = plsc.VectorSubcoreMesh(core_axis_name="c", subcore_axis_name="s")

    @functools.partial(
        pl.kernel, mesh=mesh,
        out_type=jax.ShapeDtypeStruct((B, D), jnp.float32),
        scratch_types=[
            pltpu.VMEM((b_per_w,), jnp.int32),
            pltpu.VMEM((b_per_w, D), jnp.float32),
            pltpu.SemaphoreType.DMA,
        ],
    )
    def gather_rows(idx_hbm, table_hbm, out_hbm, idx_v, rows_v, sem):
        wid = lax.axis_index("s") * info.num_cores + lax.axis_index("c")
        base = wid * b_per_w
        pltpu.sync_copy(idx_hbm.at[pl.ds(base, b_per_w)], idx_v)
        pltpu.async_copy(table_hbm.at[idx_v], rows_v, sem).wait()
        pltpu.sync_copy(rows_v, out_hbm.at[pl.ds(base, b_per_w)])

    return gather_rows


def _st_loss_body(z_ref, zq_ref, st_ref, tot_ref, acc_ref):
    m = pl.program_id(0)

    @pl.when(m == 0)
    def _init():
        acc_ref[0] = 0.0

    diff = zq_ref[...] - z_ref[...]
    st_ref[...] = z_ref[...] + diff
    acc_ref[0] += jnp.sum(diff * diff)

    @pl.when(m == pl.num_programs(0) - 1)
    def _emit():
        tot_ref[0, 0] = acc_ref[0]


def _st_and_loss(z_flat, z_q):
    rows = z_flat.shape[0]
    bm = 1024
    grid = (rows // bm,)
    return pl.pallas_call(
        _st_loss_body,
        grid=grid,
        in_specs=[
            pl.BlockSpec((bm, _E_DIM), lambda m: (m, 0)),
            pl.BlockSpec((bm, _E_DIM), lambda m: (m, 0)),
        ],
        out_specs=[
            pl.BlockSpec((bm, _E_DIM), lambda m: (m, 0)),
            pl.BlockSpec((1, 1), lambda m: (0, 0), memory_space=pltpu.SMEM),
        ],
        out_shape=[
            jax.ShapeDtypeStruct((rows, _E_DIM), jnp.float32),
            jax.ShapeDtypeStruct((1, 1), jnp.float32),
        ],
        scratch_shapes=[pltpu.SMEM((1,), jnp.float32)],
        compiler_params=pltpu.CompilerParams(
            dimension_semantics=("arbitrary",),
        ),
    )(z_flat, z_q)


def kernel(z, W):
    z_flat = z.reshape(-1, _E_DIM)
    rs = jnp.sum(z ** 2, axis=-1).reshape(-1, 1)
    cs = jnp.sum(W ** 2, axis=1)[None, :]
    z2 = (2.0 * z_flat).astype(jnp.bfloat16)
    idx2d = _dist_argmin(z2, W, rs, cs)
    idx = idx2d.reshape(-1)
    z_q = _make_sc_gather(z_flat.shape[0], _E_DIM)(idx, W)
    st, total = _st_and_loss(z_flat, z_q)
    mse = total[0, 0] / z.size
    loss = (_BETA * mse, mse)
    return st.reshape(z.shape), loss, idx


# speedup vs baseline: 1.5997x; 1.2815x over previous
import functools

import jax
import jax.numpy as jnp
from jax import lax
from jax.experimental import pallas as pl
from jax.experimental.pallas import tpu as pltpu
from jax.experimental.pallas import tpu_sc as plsc

_N_E = 8192
_E_DIM = 256
_BETA = 0.25

_BM = 512
_BN = 2048


def _dist_argmin_body(z_ref, w_ref, rs_ref, cs_ref, idx_ref, rmin_ref, ridx_ref):
    n = pl.program_id(0)
    m = pl.program_id(1)
    msl = pl.ds(m * _BM, _BM)

    @pl.when(n == 0)
    def _init():
        rmin_ref[msl, :] = jnp.full((_BM, 1), jnp.inf, jnp.float32)
        ridx_ref[msl, :] = jnp.zeros((_BM, 1), jnp.int32)

    mm2 = lax.dot_general(
        z_ref[...], w_ref[...],
        dimension_numbers=(((1,), (1,)), ((), ())),
        preferred_element_type=jnp.float32,
    )
    rs = rs_ref[...]
    v = None
    for c in range(_BN // 128):
        csc = cs_ref[:, c * 128:(c + 1) * 128]
        d2 = (rs + csc) - mm2[:, c * 128:(c + 1) * 128]
        dc = jnp.maximum(d2, 0.0)
        zdc = dc * lax.rsqrt(dc)
        if v is None:
            v = zdc
            ci = jnp.zeros((_BM, 128), jnp.float32)
        else:
            lt = zdc < v
            v = jnp.where(lt, zdc, v)
            ci = jnp.where(lt, jnp.float32(c * 128), ci)
    bmin = jnp.min(v, axis=1, keepdims=True)
    lane = lax.broadcasted_iota(jnp.int32, (_BM, 128), 1).astype(jnp.float32)
    jf = jnp.min(
        jnp.where(v == bmin, ci + lane, jnp.float32(3.0e38)),
        axis=1, keepdims=True,
    )
    bidx = n * _BN + jf.astype(jnp.int32)
    prev = rmin_ref[msl, :]
    better = bmin < prev
    bq = bmin.astype(jnp.bfloat16).astype(jnp.float32)
    rmin_ref[msl, :] = jnp.where(better, bq, prev)
    ridx_ref[msl, :] = jnp.where(better, bidx, ridx_ref[msl, :])

    @pl.when(n == pl.num_programs(0) - 1)
    def _emit():
        idx_ref[...] = ridx_ref[msl, :]


def _dist_argmin(z_flat, W, rs, cs):
    grid = (_N_E // _BN, z_flat.shape[0] // _BM)
    return pl.pallas_call(
        _dist_argmin_body,
        grid=grid,
        in_specs=[
            pl.BlockSpec((_BM, _E_DIM), lambda n, m: (m, 0)),
            pl.BlockSpec((_BN, _E_DIM), lambda n, m: (n, 0)),
            pl.BlockSpec((_BM, 1), lambda n, m: (m, 0)),
            pl.BlockSpec((1, _BN), lambda n, m: (0, n)),
        ],
        out_specs=pl.BlockSpec((_BM, 1), lambda n, m: (m, 0)),
        out_shape=jax.ShapeDtypeStruct((z_flat.shape[0], 1), jnp.int32),
        scratch_shapes=[
            pltpu.VMEM((z_flat.shape[0], 1), jnp.float32),
            pltpu.VMEM((z_flat.shape[0], 1), jnp.int32),
        ],
        compiler_params=pltpu.CompilerParams(
            dimension_semantics=("arbitrary", "arbitrary"),
        ),
    )(z_flat, W, rs, cs)


def _make_sc_gather(B, D):
    info = plsc.get_sparse_core_info()
    nw = info.num_cores * info.num_subcores
    b_per_w = B // nw
    mesh = plsc.VectorSubcoreMesh(core_axis_name="c", subcore_axis_name="s")

    @functools.partial(
        pl.kernel, mesh=mesh,
        out_type=jax.ShapeDtypeStruct((B, D), jnp.float32),
        scratch_types=[
            pltpu.VMEM((b_per_w,), jnp.int32),
            pltpu.VMEM((b_per_w, D), jnp.float32),
            pltpu.SemaphoreType.DMA,
        ],
    )
    def gather_rows(idx_hbm, table_hbm, out_hbm, idx_v, rows_v, sem):
        wid = lax.axis_index("s") * info.num_cores + lax.axis_index("c")
        base = wid * b_per_w
        pltpu.sync_copy(idx_hbm.at[pl.ds(base, b_per_w)], idx_v)
        pltpu.async_copy(table_hbm.at[idx_v], rows_v, sem).wait()
        pltpu.sync_copy(rows_v, out_hbm.at[pl.ds(base, b_per_w)])

    return gather_rows


def _st_loss_body(z_ref, zq_ref, st_ref, tot_ref, acc_ref):
    m = pl.program_id(0)

    @pl.when(m == 0)
    def _init():
        acc_ref[0] = 0.0

    diff = zq_ref[...] - z_ref[...]
    st_ref[...] = z_ref[...] + diff
    acc_ref[0] += jnp.sum(diff * diff)

    @pl.when(m == pl.num_programs(0) - 1)
    def _emit():
        tot_ref[0, 0] = acc_ref[0]


def _st_and_loss(z_flat, z_q):
    rows = z_flat.shape[0]
    bm = 1024
    grid = (rows // bm,)
    return pl.pallas_call(
        _st_loss_body,
        grid=grid,
        in_specs=[
            pl.BlockSpec((bm, _E_DIM), lambda m: (m, 0)),
            pl.BlockSpec((bm, _E_DIM), lambda m: (m, 0)),
        ],
        out_specs=[
            pl.BlockSpec((bm, _E_DIM), lambda m: (m, 0)),
            pl.BlockSpec((1, 1), lambda m: (0, 0), memory_space=pltpu.SMEM),
        ],
        out_shape=[
            jax.ShapeDtypeStruct((rows, _E_DIM), jnp.float32),
            jax.ShapeDtypeStruct((1, 1), jnp.float32),
        ],
        scratch_shapes=[pltpu.SMEM((1,), jnp.float32)],
        compiler_params=pltpu.CompilerParams(
            dimension_semantics=("arbitrary",),
        ),
    )(z_flat, z_q)


def kernel(z, W):
    z_flat = z.reshape(-1, _E_DIM)
    rs = jnp.sum(z ** 2, axis=-1).reshape(-1, 1)
    cs = jnp.sum(W ** 2, axis=1)[None, :]
    z2 = (2.0 * z_flat).astype(jnp.bfloat16)
    idx2d = _dist_argmin(z2, W, rs, cs)
    idx = idx2d.reshape(-1)
    z_q = _make_sc_gather(z_flat.shape[0], _E_DIM)(idx, W)
    st, total = _st_and_loss(z_flat, z_q)
    mse = total[0, 0] / z.size
    loss = (_BETA * mse, mse)
    return st.reshape(z.shape), loss, idx


# speedup vs baseline: 1.7111x; 1.0697x over previous
import functools

import jax
import jax.numpy as jnp
from jax import lax
from jax.experimental import pallas as pl
from jax.experimental.pallas import tpu as pltpu
from jax.experimental.pallas import tpu_sc as plsc

_N_E = 8192
_E_DIM = 256
_BETA = 0.25

_BM = 1024
_BN = 2048


def _dist_argmin_body(z_ref, w_ref, rs_ref, cs_ref, idx_ref, rmin_ref, ridx_ref):
    n = pl.program_id(0)
    m = pl.program_id(1)
    msl = pl.ds(m * _BM, _BM)

    @pl.when(n == 0)
    def _init():
        rmin_ref[msl, :] = jnp.full((_BM, 1), jnp.inf, jnp.float32)
        ridx_ref[msl, :] = jnp.zeros((_BM, 1), jnp.int32)

    mm2 = lax.dot_general(
        z_ref[...], w_ref[...],
        dimension_numbers=(((1,), (1,)), ((), ())),
        preferred_element_type=jnp.float32,
    )
    rs = rs_ref[...]
    v = None
    for c in range(_BN // 128):
        csc = cs_ref[:, c * 128:(c + 1) * 128]
        d2 = (rs + csc) - mm2[:, c * 128:(c + 1) * 128]
        dc = jnp.maximum(d2, 0.0)
        zdc = dc * lax.rsqrt(dc)
        if v is None:
            v = zdc
            ci = jnp.zeros((_BM, 128), jnp.float32)
        else:
            lt = zdc < v
            v = jnp.where(lt, zdc, v)
            ci = jnp.where(lt, jnp.float32(c * 128), ci)
    bmin = jnp.min(v, axis=1, keepdims=True)
    lane = lax.broadcasted_iota(jnp.int32, (_BM, 128), 1).astype(jnp.float32)
    jf = jnp.min(
        jnp.where(v == bmin, ci + lane, jnp.float32(3.0e38)),
        axis=1, keepdims=True,
    )
    bidx = n * _BN + jf.astype(jnp.int32)
    prev = rmin_ref[msl, :]
    better = bmin < prev
    bq = bmin.astype(jnp.bfloat16).astype(jnp.float32)
    rmin_ref[msl, :] = jnp.where(better, bq, prev)
    ridx_ref[msl, :] = jnp.where(better, bidx, ridx_ref[msl, :])

    @pl.when(n == pl.num_programs(0) - 1)
    def _emit():
        idx_ref[...] = ridx_ref[msl, :]


def _dist_argmin(z_flat, W, rs, cs):
    grid = (_N_E // _BN, z_flat.shape[0] // _BM)
    return pl.pallas_call(
        _dist_argmin_body,
        grid=grid,
        in_specs=[
            pl.BlockSpec((_BM, _E_DIM), lambda n, m: (m, 0)),
            pl.BlockSpec((_BN, _E_DIM), lambda n, m: (n, 0)),
            pl.BlockSpec((_BM, 1), lambda n, m: (m, 0)),
            pl.BlockSpec((1, _BN), lambda n, m: (0, n)),
        ],
        out_specs=pl.BlockSpec((_BM, 1), lambda n, m: (m, 0)),
        out_shape=jax.ShapeDtypeStruct((z_flat.shape[0], 1), jnp.int32),
        scratch_shapes=[
            pltpu.VMEM((z_flat.shape[0], 1), jnp.float32),
            pltpu.VMEM((z_flat.shape[0], 1), jnp.int32),
        ],
        compiler_params=pltpu.CompilerParams(
            dimension_semantics=("arbitrary", "arbitrary"),
        ),
    )(z_flat, W, rs, cs)


def _make_sc_gather(B, D):
    info = plsc.get_sparse_core_info()
    nw = info.num_cores * info.num_subcores
    b_per_w = B // nw
    mesh = plsc.VectorSubcoreMesh(core_axis_name="c", subcore_axis_name="s")

    @functools.partial(
        pl.kernel, mesh=mesh,
        out_type=jax.ShapeDtypeStruct((B, D), jnp.float32),
        scratch_types=[
            pltpu.VMEM((b_per_w,), jnp.int32),
            pltpu.VMEM((b_per_w, D), jnp.float32),
            pltpu.SemaphoreType.DMA,
        ],
    )
    def gather_rows(idx_hbm, table_hbm, out_hbm, idx_v, rows_v, sem):
        wid = lax.axis_index("s") * info.num_cores + lax.axis_index("c")
        base = wid * b_per_w
        pltpu.sync_copy(idx_hbm.at[pl.ds(base, b_per_w)], idx_v)
        pltpu.async_copy(table_hbm.at[idx_v], rows_v, sem).wait()
        pltpu.sync_copy(rows_v, out_hbm.at[pl.ds(base, b_per_w)])

    return gather_rows


def _st_loss_body(z_ref, zq_ref, st_ref, tot_ref, acc_ref):
    m = pl.program_id(0)

    @pl.when(m == 0)
    def _init():
        acc_ref[0] = 0.0

    diff = zq_ref[...] - z_ref[...]
    st_ref[...] = z_ref[...] + diff
    acc_ref[0] += jnp.sum(diff * diff)

    @pl.when(m == pl.num_programs(0) - 1)
    def _emit():
        tot_ref[0, 0] = acc_ref[0]


def _st_and_loss(z_flat, z_q):
    rows = z_flat.shape[0]
    bm = 1024
    grid = (rows // bm,)
    return pl.pallas_call(
        _st_loss_body,
        grid=grid,
        in_specs=[
            pl.BlockSpec((bm, _E_DIM), lambda m: (m, 0)),
            pl.BlockSpec((bm, _E_DIM), lambda m: (m, 0)),
        ],
        out_specs=[
            pl.BlockSpec((bm, _E_DIM), lambda m: (m, 0)),
            pl.BlockSpec((1, 1), lambda m: (0, 0), memory_space=pltpu.SMEM),
        ],
        out_shape=[
            jax.ShapeDtypeStruct((rows, _E_DIM), jnp.float32),
            jax.ShapeDtypeStruct((1, 1), jnp.float32),
        ],
        scratch_shapes=[pltpu.SMEM((1,), jnp.float32)],
        compiler_params=pltpu.CompilerParams(
            dimension_semantics=("arbitrary",),
        ),
    )(z_flat, z_q)


def kernel(z, W):
    z_flat = z.reshape(-1, _E_DIM)
    rs = jnp.sum(z ** 2, axis=-1).reshape(-1, 1)
    cs = jnp.sum(W ** 2, axis=1)[None, :]
    z2 = (2.0 * z_flat).astype(jnp.bfloat16)
    idx2d = _dist_argmin(z2, W, rs, cs)
    idx = idx2d.reshape(-1)
    z_q = _make_sc_gather(z_flat.shape[0], _E_DIM)(idx, W)
    st, total = _st_and_loss(z_flat, z_q)
    mse = total[0, 0] / z.size
    loss = (_BETA * mse, mse)
    return st.reshape(z.shape), loss, idx


# speedup vs baseline: 1.8495x; 1.0809x over previous
import functools

import jax
import jax.numpy as jnp
from jax import lax
from jax.experimental import pallas as pl
from jax.experimental.pallas import tpu as pltpu
from jax.experimental.pallas import tpu_sc as plsc

_N_E = 8192
_E_DIM = 256
_BETA = 0.25

_BM = 2048
_BN = 2048


def _dist_argmin_body(z_ref, w_ref, rs_ref, cs_ref, idx_ref, rmin_ref, ridx_ref):
    n = pl.program_id(0)
    m = pl.program_id(1)
    msl = pl.ds(m * _BM, _BM)

    @pl.when(n == 0)
    def _init():
        rmin_ref[msl, :] = jnp.full((_BM, 1), jnp.inf, jnp.float32)
        ridx_ref[msl, :] = jnp.zeros((_BM, 1), jnp.int32)

    mm2 = lax.dot_general(
        z_ref[...], w_ref[...],
        dimension_numbers=(((1,), (1,)), ((), ())),
        preferred_element_type=jnp.float32,
    )
    rs = rs_ref[...]
    v = None
    for c in range(_BN // 128):
        csc = cs_ref[:, c * 128:(c + 1) * 128]
        d2 = (rs + csc) - mm2[:, c * 128:(c + 1) * 128]
        dc = jnp.maximum(d2, 0.0)
        zdc = dc * lax.rsqrt(dc)
        if v is None:
            v = zdc
            ci = jnp.zeros((_BM, 128), jnp.float32)
        else:
            lt = zdc < v
            v = jnp.where(lt, zdc, v)
            ci = jnp.where(lt, jnp.float32(c * 128), ci)
    bmin = jnp.min(v, axis=1, keepdims=True)
    lane = lax.broadcasted_iota(jnp.int32, (_BM, 128), 1).astype(jnp.float32)
    jf = jnp.min(
        jnp.where(v == bmin, ci + lane, jnp.float32(3.0e38)),
        axis=1, keepdims=True,
    )
    bidx = n * _BN + jf.astype(jnp.int32)
    prev = rmin_ref[msl, :]
    better = bmin < prev
    bq = bmin.astype(jnp.bfloat16).astype(jnp.float32)
    rmin_ref[msl, :] = jnp.where(better, bq, prev)
    ridx_ref[msl, :] = jnp.where(better, bidx, ridx_ref[msl, :])

    @pl.when(n == pl.num_programs(0) - 1)
    def _emit():
        idx_ref[...] = ridx_ref[msl, :]


def _dist_argmin(z_flat, W, rs, cs):
    grid = (_N_E // _BN, z_flat.shape[0] // _BM)
    return pl.pallas_call(
        _dist_argmin_body,
        grid=grid,
        in_specs=[
            pl.BlockSpec((_BM, _E_DIM), lambda n, m: (m, 0)),
            pl.BlockSpec((_BN, _E_DIM), lambda n, m: (n, 0)),
            pl.BlockSpec((_BM, 1), lambda n, m: (m, 0)),
            pl.BlockSpec((1, _BN), lambda n, m: (0, n)),
        ],
        out_specs=pl.BlockSpec((_BM, 1), lambda n, m: (m, 0)),
        out_shape=jax.ShapeDtypeStruct((z_flat.shape[0], 1), jnp.int32),
        scratch_shapes=[
            pltpu.VMEM((z_flat.shape[0], 1), jnp.float32),
            pltpu.VMEM((z_flat.shape[0], 1), jnp.int32),
        ],
        compiler_params=pltpu.CompilerParams(
            dimension_semantics=("arbitrary", "arbitrary"),
        ),
    )(z_flat, W, rs, cs)


def _make_sc_gather(B, D):
    info = plsc.get_sparse_core_info()
    nw = info.num_cores * info.num_subcores
    b_per_w = B // nw
    mesh = plsc.VectorSubcoreMesh(core_axis_name="c", subcore_axis_name="s")

    @functools.partial(
        pl.kernel, mesh=mesh,
        out_type=jax.ShapeDtypeStruct((B, D), jnp.float32),
        scratch_types=[
            pltpu.VMEM((b_per_w,), jnp.int32),
            pltpu.VMEM((b_per_w, D), jnp.float32),
            pltpu.SemaphoreType.DMA,
        ],
    )
    def gather_rows(idx_hbm, table_hbm, out_hbm, idx_v, rows_v, sem):
        wid = lax.axis_index("s") * info.num_cores + lax.axis_index("c")
        base = wid * b_per_w
        pltpu.sync_copy(idx_hbm.at[pl.ds(base, b_per_w)], idx_v)
        pltpu.async_copy(table_hbm.at[idx_v], rows_v, sem).wait()
        pltpu.sync_copy(rows_v, out_hbm.at[pl.ds(base, b_per_w)])

    return gather_rows


def _st_loss_body(z_ref, zq_ref, st_ref, tot_ref, acc_ref):
    m = pl.program_id(0)

    @pl.when(m == 0)
    def _init():
        acc_ref[0] = 0.0

    diff = zq_ref[...] - z_ref[...]
    st_ref[...] = z_ref[...] + diff
    acc_ref[0] += jnp.sum(diff * diff)

    @pl.when(m == pl.num_programs(0) - 1)
    def _emit():
        tot_ref[0, 0] = acc_ref[0]


def _st_and_loss(z_flat, z_q):
    rows = z_flat.shape[0]
    bm = 1024
    grid = (rows // bm,)
    return pl.pallas_call(
        _st_loss_body,
        grid=grid,
        in_specs=[
            pl.BlockSpec((bm, _E_DIM), lambda m: (m, 0)),
            pl.BlockSpec((bm, _E_DIM), lambda m: (m, 0)),
        ],
        out_specs=[
            pl.BlockSpec((bm, _E_DIM), lambda m: (m, 0)),
            pl.BlockSpec((1, 1), lambda m: (0, 0), memory_space=pltpu.SMEM),
        ],
        out_shape=[
            jax.ShapeDtypeStruct((rows, _E_DIM), jnp.float32),
            jax.ShapeDtypeStruct((1, 1), jnp.float32),
        ],
        scratch_shapes=[pltpu.SMEM((1,), jnp.float32)],
        compiler_params=pltpu.CompilerParams(
            dimension_semantics=("arbitrary",),
        ),
    )(z_flat, z_q)


def kernel(z, W):
    z_flat = z.reshape(-1, _E_DIM)
    rs = jnp.sum(z ** 2, axis=-1).reshape(-1, 1)
    cs = jnp.sum(W ** 2, axis=1)[None, :]
    z2 = (2.0 * z_flat).astype(jnp.bfloat16)
    idx2d = _dist_argmin(z2, W, rs, cs)
    idx = idx2d.reshape(-1)
    z_q = _make_sc_gather(z_flat.shape[0], _E_DIM)(idx, W)
    st, total = _st_and_loss(z_flat, z_q)
    mse = total[0, 0] / z.size
    loss = (_BETA * mse, mse)
    return st.reshape(z.shape), loss, idx


# speedup vs baseline: 1.9078x; 1.0315x over previous
import functools

import jax
import jax.numpy as jnp
from jax import lax
from jax.experimental import pallas as pl
from jax.experimental.pallas import tpu as pltpu
from jax.experimental.pallas import tpu_sc as plsc

_N_E = 8192
_E_DIM = 256
_BETA = 0.25

_BM = 4096
_BN = 2048


def _dist_argmin_body(z_ref, w_ref, rs_ref, cs_ref, idx_ref, rmin_ref, ridx_ref):
    n = pl.program_id(0)
    m = pl.program_id(1)
    msl = pl.ds(m * _BM, _BM)

    @pl.when(n == 0)
    def _init():
        rmin_ref[msl, :] = jnp.full((_BM, 1), jnp.inf, jnp.float32)
        ridx_ref[msl, :] = jnp.zeros((_BM, 1), jnp.int32)

    mm2 = lax.dot_general(
        z_ref[...], w_ref[...],
        dimension_numbers=(((1,), (1,)), ((), ())),
        preferred_element_type=jnp.float32,
    )
    rs = rs_ref[...]
    v = None
    for c in range(_BN // 128):
        csc = cs_ref[:, c * 128:(c + 1) * 128]
        d2 = (rs + csc) - mm2[:, c * 128:(c + 1) * 128]
        dc = jnp.maximum(d2, 0.0)
        zdc = dc * lax.rsqrt(dc)
        if v is None:
            v = zdc
            ci = jnp.zeros((_BM, 128), jnp.float32)
        else:
            lt = zdc < v
            v = jnp.where(lt, zdc, v)
            ci = jnp.where(lt, jnp.float32(c * 128), ci)
    bmin = jnp.min(v, axis=1, keepdims=True)
    lane = lax.broadcasted_iota(jnp.int32, (_BM, 128), 1).astype(jnp.float32)
    jf = jnp.min(
        jnp.where(v == bmin, ci + lane, jnp.float32(3.0e38)),
        axis=1, keepdims=True,
    )
    bidx = n * _BN + jf.astype(jnp.int32)
    prev = rmin_ref[msl, :]
    better = bmin < prev
    bq = bmin.astype(jnp.bfloat16).astype(jnp.float32)
    rmin_ref[msl, :] = jnp.where(better, bq, prev)
    ridx_ref[msl, :] = jnp.where(better, bidx, ridx_ref[msl, :])

    @pl.when(n == pl.num_programs(0) - 1)
    def _emit():
        idx_ref[...] = ridx_ref[msl, :]


def _dist_argmin(z_flat, W, rs, cs):
    grid = (_N_E // _BN, z_flat.shape[0] // _BM)
    return pl.pallas_call(
        _dist_argmin_body,
        grid=grid,
        in_specs=[
            pl.BlockSpec((_BM, _E_DIM), lambda n, m: (m, 0)),
            pl.BlockSpec((_BN, _E_DIM), lambda n, m: (n, 0)),
            pl.BlockSpec((_BM, 1), lambda n, m: (m, 0)),
            pl.BlockSpec((1, _BN), lambda n, m: (0, n)),
        ],
        out_specs=pl.BlockSpec((_BM, 1), lambda n, m: (m, 0)),
        out_shape=jax.ShapeDtypeStruct((z_flat.shape[0], 1), jnp.int32),
        scratch_shapes=[
            pltpu.VMEM((z_flat.shape[0], 1), jnp.float32),
            pltpu.VMEM((z_flat.shape[0], 1), jnp.int32),
        ],
        compiler_params=pltpu.CompilerParams(
            dimension_semantics=("arbitrary", "arbitrary"),
        ),
    )(z_flat, W, rs, cs)


def _make_sc_gather(B, D):
    info = plsc.get_sparse_core_info()
    nw = info.num_cores * info.num_subcores
    b_per_w = B // nw
    mesh = plsc.VectorSubcoreMesh(core_axis_name="c", subcore_axis_name="s")

    @functools.partial(
        pl.kernel, mesh=mesh,
        out_type=jax.ShapeDtypeStruct((B, D), jnp.float32),
        scratch_types=[
            pltpu.VMEM((b_per_w,), jnp.int32),
            pltpu.VMEM((b_per_w, D), jnp.float32),
            pltpu.SemaphoreType.DMA,
        ],
    )
    def gather_rows(idx_hbm, table_hbm, out_hbm, idx_v, rows_v, sem):
        wid = lax.axis_index("s") * info.num_cores + lax.axis_index("c")
        base = wid * b_per_w
        pltpu.sync_copy(idx_hbm.at[pl.ds(base, b_per_w)], idx_v)
        pltpu.async_copy(table_hbm.at[idx_v], rows_v, sem).wait()
        pltpu.sync_copy(rows_v, out_hbm.at[pl.ds(base, b_per_w)])

    return gather_rows


def _st_loss_body(z_ref, zq_ref, st_ref, tot_ref, acc_ref):
    m = pl.program_id(0)

    @pl.when(m == 0)
    def _init():
        acc_ref[0] = 0.0

    diff = zq_ref[...] - z_ref[...]
    st_ref[...] = z_ref[...] + diff
    acc_ref[0] += jnp.sum(diff * diff)

    @pl.when(m == pl.num_programs(0) - 1)
    def _emit():
        tot_ref[0, 0] = acc_ref[0]


def _st_and_loss(z_flat, z_q):
    rows = z_flat.shape[0]
    bm = 1024
    grid = (rows // bm,)
    return pl.pallas_call(
        _st_loss_body,
        grid=grid,
        in_specs=[
            pl.BlockSpec((bm, _E_DIM), lambda m: (m, 0)),
            pl.BlockSpec((bm, _E_DIM), lambda m: (m, 0)),
        ],
        out_specs=[
            pl.BlockSpec((bm, _E_DIM), lambda m: (m, 0)),
            pl.BlockSpec((1, 1), lambda m: (0, 0), memory_space=pltpu.SMEM),
        ],
        out_shape=[
            jax.ShapeDtypeStruct((rows, _E_DIM), jnp.float32),
            jax.ShapeDtypeStruct((1, 1), jnp.float32),
        ],
        scratch_shapes=[pltpu.SMEM((1,), jnp.float32)],
        compiler_params=pltpu.CompilerParams(
            dimension_semantics=("arbitrary",),
        ),
    )(z_flat, z_q)


def kernel(z, W):
    z_flat = z.reshape(-1, _E_DIM)
    rs = jnp.sum(z ** 2, axis=-1).reshape(-1, 1)
    cs = jnp.sum(W ** 2, axis=1)[None, :]
    z2 = (2.0 * z_flat).astype(jnp.bfloat16)
    idx2d = _dist_argmin(z2, W, rs, cs)
    idx = idx2d.reshape(-1)
    z_q = _make_sc_gather(z_flat.shape[0], _E_DIM)(idx, W)
    st, total = _st_and_loss(z_flat, z_q)
    mse = total[0, 0] / z.size
    loss = (_BETA * mse, mse)
    return st.reshape(z.shape), loss, idx
